# baseline (speedup 1.0000x reference)
import functools

import jax
import jax.numpy as jnp
from jax import lax
from jax.experimental import pallas as pl
from jax.experimental.pallas import tpu as pltpu
from jax.experimental.pallas import tpu_sc as plsc

VOCAB = 1000000
D_MODEL = 64
SCALE = 8.0
SEQ = 16384
POS = 50
NTOK = SEQ * POS

NUM_CORES = 2
NUM_SUBCORES = 16
NW = NUM_CORES * NUM_SUBCORES

B_W = SEQ // NW
SUB = 128
NSUB = B_W // SUB
HPOS = POS // 2
NCHUNK = HPOS * NSUB

TBLK = 4096
TGRID = (VOCAB + TBLK - 1) // TBLK

OBLK = 2048
OGRID = SEQ // OBLK


def _tin_body(t_ref, o_ref):
    blk = t_ref[...]
    blk_t = jnp.transpose(blk, (1, 0))
    o_ref[...] = jnp.concatenate(
        [blk_t, jnp.zeros((TBLK, 128 - D_MODEL), jnp.float32)], axis=1)


def _tc_transpose_in(tt):
    return pl.pallas_call(
        _tin_body,
        grid=(TGRID,),
        in_specs=[pl.BlockSpec((D_MODEL, TBLK), lambda i: (0, i))],
        out_specs=pl.BlockSpec((TBLK, 128), lambda i: (i, 0)),
        out_shape=jax.ShapeDtypeStruct((TGRID * TBLK, 128), jnp.float32),
    )(tt)


def _tout_body(r_ref, o_ref):
    blk = r_ref[:, :D_MODEL]
    o_ref[0] = jnp.transpose(blk, (1, 0)) * SCALE


def _tc_transpose_out(rows, pos):
    return pl.pallas_call(
        _tout_body,
        grid=(pos, OGRID),
        in_specs=[pl.BlockSpec((OBLK, 128),
                               lambda s, j: (s * OGRID + j, 0))],
        out_specs=pl.BlockSpec((1, D_MODEL, OBLK), lambda s, j: (s, 0, j)),
        out_shape=jax.ShapeDtypeStruct((pos, D_MODEL, SEQ), jnp.float32),
    )(rows)


_mesh = plsc.VectorSubcoreMesh(core_axis_name="c", subcore_axis_name="s")


@functools.partial(
    pl.kernel,
    mesh=_mesh,
    out_type=jax.ShapeDtypeStruct((HPOS * SEQ, 128), jnp.float32),
    compiler_params=pltpu.CompilerParams(
        use_tc_tiling_on_sc=True, needs_layout_passes=False),
    scratch_types=[
        pltpu.VMEM((HPOS * B_W,), jnp.int32),
        pltpu.VMEM((4, SUB, 128), jnp.float32),
        pltpu.SemaphoreType.DMA,
        pltpu.SemaphoreType.DMA,
        pltpu.SemaphoreType.DMA,
        pltpu.SemaphoreType.DMA,
        pltpu.SemaphoreType.DMA,
        pltpu.SemaphoreType.DMA,
        pltpu.SemaphoreType.DMA,
        pltpu.SemaphoreType.DMA,
    ],
)
def _sc_gather(xs_hbm, trm_hbm, out_hbm, idx_v, gbuf,
               gsem0, gsem1, gsem2, gsem3, ssem0, ssem1, ssem2, ssem3):
    gsems = (gsem0, gsem1, gsem2, gsem3)
    ssems = (ssem0, ssem1, ssem2, ssem3)
    wid = lax.axis_index("s") * NUM_CORES + lax.axis_index("c")
    b0w = wid * B_W

    def stage(s, c):
        pltpu.sync_copy(xs_hbm.at[pl.ds(s * SEQ + b0w, B_W)],
                        idx_v.at[pl.ds(s * B_W, B_W)])
        return c
    lax.fori_loop(0, HPOS, stage, 0)

    def out_row(chunk):
        s = chunk // NSUB
        sub = chunk - s * NSUB
        return s * SEQ + b0w + sub * SUB

    def fire_gather(chunk, slot):
        pltpu.async_copy(
            trm_hbm.at[idx_v.at[pl.ds(chunk * SUB, SUB)]],
            gbuf.at[slot],
            gsems[slot],
        )

    def wait_gather(slot):
        pltpu.make_async_copy(
            trm_hbm.at[idx_v.at[pl.ds(0, SUB)]],
            gbuf.at[slot],
            gsems[slot],
        ).wait()

    def fire_scatter(chunk, slot):
        pltpu.async_copy(gbuf.at[slot],
                         out_hbm.at[pl.ds(out_row(chunk), SUB)],
                         ssems[slot])

    def wait_scatter(chunk, slot):
        pltpu.make_async_copy(gbuf.at[slot],
                              out_hbm.at[pl.ds(out_row(chunk), SUB)],
                              ssems[slot]).wait()

    fire_gather(jnp.int32(0), 0)
    fire_gather(jnp.int32(1), 1)

    def body(t, carry):
        for p in range(4):
            slot = p
            c = t * 4 + p
            wait_gather(slot)
            fire_scatter(c, slot)

            nslot = (p + 2) % 4

            @pl.when(c >= 2)
            def _():
                wait_scatter(c - 2, nslot)

            @pl.when(c + 2 < NCHUNK)
            def _():
                fire_gather(c + 2, nslot)
        return carry

    lax.fori_loop(0, NCHUNK // 4, body, 0)

    wait_scatter(NCHUNK - 2, 2)
    wait_scatter(NCHUNK - 1, 3)


def kernel(x, table):
    tt = table.T
    t_rm = _tc_transpose_in(tt)
    xs = x.T.reshape(-1).astype(jnp.int32)
    half = HPOS * SEQ
    rows_a = _sc_gather(xs[:half], t_rm)
    rows_b = _sc_gather(xs[half:], t_rm)
    out_a = _tc_transpose_out(rows_a, HPOS)
    out_b = _tc_transpose_out(rows_b, HPOS)
    out3 = jnp.concatenate([out_a, out_b], axis=0)
    return jnp.transpose(out3, (2, 0, 1))


# speedup vs baseline: 1.2695x; 1.2695x over previous
import functools

import jax
import jax.numpy as jnp
from jax import lax
from jax.experimental import pallas as pl
from jax.experimental.pallas import tpu as pltpu
from jax.experimental.pallas import tpu_sc as plsc

VOCAB = 1000000
D_MODEL = 64
SCALE = 8.0
SEQ = 16384
POS = 50
NTOK = SEQ * POS

NUM_CORES = 2
NUM_SUBCORES = 16
NW = NUM_CORES * NUM_SUBCORES

B_W = SEQ // NW
SUB = 128
NSUB = B_W // SUB
NCHUNK = POS * NSUB

TBLK = 8192
TGRID = (VOCAB + TBLK - 1) // TBLK

OBLK = 4096
OGRID = SEQ // OBLK


def _tin_body(t_ref, o_ref):
    blk = t_ref[...]
    blk_t = jnp.transpose(blk, (1, 0))
    o_ref[...] = jnp.concatenate(
        [blk_t, jnp.zeros((TBLK, 128 - D_MODEL), jnp.float32)], axis=1)


def _tc_transpose_in(tt):
    return pl.pallas_call(
        _tin_body,
        grid=(TGRID,),
        in_specs=[pl.BlockSpec((D_MODEL, TBLK), lambda i: (0, i))],
        out_specs=pl.BlockSpec((TBLK, 128), lambda i: (i, 0)),
        out_shape=jax.ShapeDtypeStruct((TGRID * TBLK, 128), jnp.float32),
    )(tt)


def _tout_body(r_ref, o_ref):
    blk = r_ref[:, :D_MODEL]
    o_ref[0] = jnp.transpose(blk, (1, 0)) * SCALE


def _tc_transpose_out(rows, pos):
    return pl.pallas_call(
        _tout_body,
        grid=(pos, OGRID),
        in_specs=[pl.BlockSpec((OBLK, 128),
                               lambda s, j: (s * OGRID + j, 0))],
        out_specs=pl.BlockSpec((1, D_MODEL, OBLK), lambda s, j: (s, 0, j)),
        out_shape=jax.ShapeDtypeStruct((pos, D_MODEL, SEQ), jnp.float32),
    )(rows)


_mesh = plsc.VectorSubcoreMesh(core_axis_name="c", subcore_axis_name="s")


@functools.partial(
    pl.kernel,
    mesh=_mesh,
    out_type=jax.ShapeDtypeStruct((NTOK, 128), jnp.float32),
    compiler_params=pltpu.CompilerParams(
        use_tc_tiling_on_sc=True, needs_layout_passes=False),
    scratch_types=[
        pltpu.VMEM((POS * B_W,), jnp.int32),
        pltpu.VMEM((4, SUB, 128), jnp.float32),
        pltpu.SemaphoreType.DMA,
        pltpu.SemaphoreType.DMA,
        pltpu.SemaphoreType.DMA,
        pltpu.SemaphoreType.DMA,
        pltpu.SemaphoreType.DMA,
        pltpu.SemaphoreType.DMA,
        pltpu.SemaphoreType.DMA,
        pltpu.SemaphoreType.DMA,
    ],
)
def _sc_gather(xs_hbm, trm_hbm, out_hbm, idx_v, gbuf,
               gsem0, gsem1, gsem2, gsem3, ssem0, ssem1, ssem2, ssem3):
    gsems = (gsem0, gsem1, gsem2, gsem3)
    ssems = (ssem0, ssem1, ssem2, ssem3)
    wid = lax.axis_index("s") * NUM_CORES + lax.axis_index("c")
    b0w = wid * B_W

    def stage(s, c):
        pltpu.sync_copy(xs_hbm.at[pl.ds(s * SEQ + b0w, B_W)],
                        idx_v.at[pl.ds(s * B_W, B_W)])
        return c
    lax.fori_loop(0, POS, stage, 0)

    def out_row(chunk):
        s = chunk // NSUB
        sub = chunk - s * NSUB
        return s * SEQ + b0w + sub * SUB

    def fire_gather(chunk, slot):
        pltpu.async_copy(
            trm_hbm.at[idx_v.at[pl.ds(chunk * SUB, SUB)]],
            gbuf.at[slot],
            gsems[slot],
        )

    def wait_gather(slot):
        pltpu.make_async_copy(
            trm_hbm.at[idx_v.at[pl.ds(0, SUB)]],
            gbuf.at[slot],
            gsems[slot],
        ).wait()

    def fire_scatter(chunk, slot):
        pltpu.async_copy(gbuf.at[slot],
                         out_hbm.at[pl.ds(out_row(chunk), SUB)],
                         ssems[slot])

    def wait_scatter(chunk, slot):
        pltpu.make_async_copy(gbuf.at[slot],
                              out_hbm.at[pl.ds(out_row(chunk), SUB)],
                              ssems[slot]).wait()

    fire_gather(jnp.int32(0), 0)
    fire_gather(jnp.int32(1), 1)

    def body(t, carry):
        for p in range(4):
            slot = p
            c = t * 4 + p
            wait_gather(slot)
            fire_scatter(c, slot)

            nslot = (p + 2) % 4

            @pl.when(c >= 2)
            def _():
                wait_scatter(c - 2, nslot)

            @pl.when(c + 2 < NCHUNK)
            def _():
                fire_gather(c + 2, nslot)
        return carry

    lax.fori_loop(0, NCHUNK // 4, body, 0)

    wait_scatter(NCHUNK - 2, 2)
    wait_scatter(NCHUNK - 1, 3)


def kernel(x, table):
    tt = table.T
    t_rm = _tc_transpose_in(tt)
    xs = x.T.reshape(-1).astype(jnp.int32)
    rows = _sc_gather(xs, t_rm)
    out3 = _tc_transpose_out(rows, POS)
    return jnp.transpose(out3, (2, 0, 1))
